# baseline (speedup 1.0000x reference)
import dataclasses

import jax
import jax.numpy as jnp
from jax import lax
from jax.experimental import pallas as pl
from jax.experimental.pallas import tpu as pltpu
from jax.experimental.pallas import tpu_sc as plsc

N = 10000
E = 320000
D = 128

NC = 2
NS = 16
NW = NC * NS
L = 16

B = 128
RING = 8
NCH0 = 15
NCH1 = 5
CHE = RING * B
TOTCH = NS * (NCH0 + NCH1)
EPAD = TOTCH * CHE

NPAD = 10112
RPS = NPAD // NS
RBLK = 1000


def _sc_params():
    cp = pltpu.CompilerParams()
    if "needs_layout_passes" in pltpu.CompilerParams.__dataclass_fields__:
        cp = dataclasses.replace(cp, needs_layout_passes=False)
    return cp


def _proj_body(x_ref, w_ref, a_ref, z_ref, s1_ref, s2_ref):
    z = lax.dot_general(x_ref[...], w_ref[...], (((1,), (1,)), ((), ())),
                        preferred_element_type=jnp.float32)
    z_ref[0] = z
    z_ref[1] = z
    s = jnp.dot(z, a_ref[...], preferred_element_type=jnp.float32)
    s1_ref[...] = s[:, 0].reshape(1, 1, RBLK)
    s2_ref[...] = s[:, 1].reshape(1, 1, RBLK)


def _proj(x, W, A):
    return pl.pallas_call(
        _proj_body,
        grid=(N // RBLK,),
        in_specs=[
            pl.BlockSpec((RBLK, D), lambda i: (i, 0)),
            pl.BlockSpec((D, D), lambda i: (0, 0)),
            pl.BlockSpec((D, 2), lambda i: (0, 0)),
        ],
        out_specs=[
            pl.BlockSpec((NC, RBLK, D), lambda i: (0, i, 0)),
            pl.BlockSpec((1, 1, RBLK), lambda i: (i, 0, 0)),
            pl.BlockSpec((1, 1, RBLK), lambda i: (i, 0, 0)),
        ],
        out_shape=[
            jax.ShapeDtypeStruct((NC, N, D), jnp.float32),
            jax.ShapeDtypeStruct((N // RBLK, 1, RBLK), jnp.float32),
            jax.ShapeDtypeStruct((N // RBLK, 1, RBLK), jnp.float32),
        ],
    )(x, W, A)


def _wpass_body(s1_hbm, s2_hbm, src_hbm, dst_hbm, w_hbm, den_hbm,
                s1_v, s2_v, src_v, dst_v, w_st, den_part):
    cid = lax.axis_index("c")
    sid = lax.axis_index("s")
    wid = sid * NC + cid
    nch = jnp.where(cid == 0, NCH0, NCH1)
    st = jnp.where(cid == 0, sid * NCH0, NS * NCH0 + sid * NCH1)

    pltpu.sync_copy(s1_hbm, s1_v)
    pltpu.sync_copy(s2_hbm, s2_v)

    @pl.loop(0, N // L)
    def _(i):
        off = pl.multiple_of(i * L, L)
        den_part[pl.ds(off, L)] = jnp.zeros((L,), jnp.float32)

    @pl.loop(0, nch)
    def _(c):
        ch = st + c
        pltpu.sync_copy(src_hbm.at[ch], src_v)
        pltpu.sync_copy(dst_hbm.at[ch], dst_v)

        @pl.loop(0, RING)
        def _(b):
            for g in range(B // L):
                sv = src_v[b, pl.ds(g * L, L)]
                dv = dst_v[b, pl.ds(g * L, L)]
                e = plsc.load_gather(s1_v, [sv]) + plsc.load_gather(s2_v, [dv])
                e = jnp.where(e > 0, e, e * 0.01)
                w = jnp.exp(e)
                gid = ch * CHE + b * B + g * L + lax.iota(jnp.int32, L)
                w = jnp.where(gid < E, w, 0.0)
                w_st[b, pl.ds(g * L, L)] = w
                plsc.addupdate_scatter(den_part, [dv], w)

        pltpu.sync_copy(w_st, w_hbm.at[ch])

    pltpu.sync_copy(den_part, den_hbm.at[wid])


def _wpass(s1, s2, src_p, dst_p):
    mesh = plsc.VectorSubcoreMesh(core_axis_name="c", subcore_axis_name="s")
    kern = pl.kernel(
        _wpass_body,
        out_type=[
            jax.ShapeDtypeStruct((TOTCH, RING, B), jnp.float32),
            jax.ShapeDtypeStruct((NW, N), jnp.float32),
        ],
        mesh=mesh,
        scratch_types=[
            pltpu.VMEM((N,), jnp.float32),
            pltpu.VMEM((N,), jnp.float32),
            pltpu.VMEM((RING, B), jnp.int32),
            pltpu.VMEM((RING, B), jnp.int32),
            pltpu.VMEM((RING, B), jnp.float32),
            pltpu.VMEM((N,), jnp.float32),
        ],
        compiler_params=_sc_params(),
    )
    return kern(s1, s2, src_p, dst_p)


def _acc_body(z_hbm, w_hbm, src_hbm, dst_hbm, num_hbm,
              src_v, dst_v, w_v, rows, num_acc, gsem, ssem):
    cid = lax.axis_index("c")
    sid = lax.axis_index("s")
    nch = jnp.where(cid == 0, NCH0, NCH1)
    st = jnp.where(cid == 0, sid * NCH0, NS * NCH0 + sid * NCH1)

    @pl.loop(0, B)
    def _(r):
        for k in range(D // L):
            rows[0, r, pl.ds(k * L, L)] = jnp.zeros((L,), jnp.float32)

    for k in range(4):
        pltpu.sync_copy(rows.at[0], num_acc.at[pl.ds(sid * RPS + k * B, B)])
    pltpu.sync_copy(rows.at[0].at[pl.ds(0, RPS - 4 * B)],
                    num_acc.at[pl.ds(sid * RPS + 4 * B, RPS - 4 * B)])
    plsc.subcore_barrier()

    def _scale(b, cur):
        bvec = jnp.full((L,), b, jnp.int32)
        for r in range(B):
            wr = plsc.load_gather(w_v, [bvec, jnp.full((L,), r, jnp.int32)])
            for k in range(D // L):
                sl = pl.ds(k * L, L)
                rows[cur, r, sl] = rows[cur, r, sl] * wr

    def _scatter_wait(b, cur):
        pltpu.make_async_copy(rows.at[cur], num_acc.at[dst_v.at[b]],
                              ssem).wait()

    @pl.loop(0, nch)
    def _(c):
        ch = st + c
        pltpu.sync_copy(src_hbm.at[ch], src_v)
        pltpu.sync_copy(dst_hbm.at[ch], dst_v)
        pltpu.sync_copy(w_hbm.at[ch], w_v)

        pltpu.sync_copy(z_hbm.at[cid].at[src_v.at[0]], rows.at[0])

        @pl.loop(0, RING - 1)
        def _(b):
            cur = b & 1
            nxt = (b + 1) & 1

            @pl.when(b >= 1)
            def _():
                _scatter_wait(b - 1, nxt)

            h = pltpu.async_copy(z_hbm.at[cid].at[src_v.at[b + 1]], rows.at[nxt],
                                 gsem)
            _scale(b, cur)
            pltpu.async_copy(rows.at[cur], num_acc.at[dst_v.at[b]], ssem,
                             add=True)
            h.wait()

        _scale(RING - 1, (RING - 1) & 1)
        pltpu.async_copy(rows.at[(RING - 1) & 1],
                         num_acc.at[dst_v.at[RING - 1]], ssem, add=True)
        _scatter_wait(RING - 2, (RING - 2) & 1)
        _scatter_wait(RING - 1, (RING - 1) & 1)

    plsc.subcore_barrier()
    pltpu.sync_copy(num_acc.at[pl.ds(sid * RPS, RPS)],
                    num_hbm.at[cid].at[pl.ds(sid * RPS, RPS)])


def _acc(z, w, src_p, dst_p):
    mesh = plsc.VectorSubcoreMesh(core_axis_name="c", subcore_axis_name="s")
    kern = pl.kernel(
        _acc_body,
        out_type=jax.ShapeDtypeStruct((NC, NPAD, D), jnp.float32),
        mesh=mesh,
        scratch_types=[
            pltpu.VMEM((RING, B), jnp.int32),
            pltpu.VMEM((RING, B), jnp.int32),
            pltpu.VMEM((RING, B), jnp.float32),
            pltpu.VMEM((2, B, D), jnp.float32),
            pltpu.VMEM_SHARED((NPAD, D), jnp.float32),
            pltpu.SemaphoreType.DMA,
            pltpu.SemaphoreType.DMA,
        ],
        compiler_params=_sc_params(),
    )
    return kern(z, w, src_p, dst_p)


def _div_body(num_ref, den_ref, h_ref):
    nsum = num_ref[0] + num_ref[1]
    d = jnp.sum(den_ref[:, 0, 0, :], axis=0)
    dsafe = jnp.where(d > 0, d, 1.0)
    h_ref[...] = jnp.where(d[:, None] > 0, nsum / dsafe[:, None], 0.0)


def _div(num, den):
    return pl.pallas_call(
        _div_body,
        grid=(N // RBLK,),
        in_specs=[
            pl.BlockSpec((NC, RBLK, D), lambda i: (0, i, 0)),
            pl.BlockSpec((NW, 1, 1, RBLK), lambda i: (0, i, 0, 0)),
        ],
        out_specs=pl.BlockSpec((RBLK, D), lambda i: (i, 0)),
        out_shape=jax.ShapeDtypeStruct((N, D), jnp.float32),
    )(num, den)


def kernel(x, edge_index, W, attn_w):
    src = edge_index[0].astype(jnp.int32)
    dst = edge_index[1].astype(jnp.int32)
    pad = EPAD - E
    src_p = jnp.pad(src, (0, pad)).reshape(TOTCH, RING, B)
    dst_p = jnp.pad(dst, (0, pad)).reshape(TOTCH, RING, B)
    A = jnp.stack([attn_w[:D], attn_w[D:]], axis=1)
    z, s1, s2 = _proj(x, W, A)
    s1 = s1.reshape(N)
    s2 = s2.reshape(N)
    w, den = _wpass(s1, s2, src_p, dst_p)
    num = _acc(z, w, src_p, dst_p)
    den4 = den.reshape(NW, N // RBLK, 1, RBLK)
    return _div(num, den4)


# speedup vs baseline: 1.0735x; 1.0735x over previous
import dataclasses

import jax
import jax.numpy as jnp
from jax import lax
from jax.experimental import pallas as pl
from jax.experimental.pallas import tpu as pltpu
from jax.experimental.pallas import tpu_sc as plsc

N = 10000
E = 320000
D = 128

NC = 2
NS = 16
NW = NC * NS
L = 16

B = 128
RING = 16
NCH0 = 5
NCH1 = 5
CHE = RING * B
TOTCH = NS * (NCH0 + NCH1)
EPAD = TOTCH * CHE

NPAD = 10112
RPS = NPAD // NS
RBLK = 1000


def _sc_params():
    cp = pltpu.CompilerParams()
    if "needs_layout_passes" in pltpu.CompilerParams.__dataclass_fields__:
        cp = dataclasses.replace(cp, needs_layout_passes=False)
    return cp


def _proj_body(x_ref, w_ref, a_ref, z_ref, s1_ref, s2_ref):
    z = lax.dot_general(x_ref[...], w_ref[...], (((1,), (1,)), ((), ())),
                        preferred_element_type=jnp.float32)
    z_ref[...] = z
    s = jnp.dot(z, a_ref[...], preferred_element_type=jnp.float32)
    s1_ref[...] = s[:, 0].reshape(1, 1, RBLK)
    s2_ref[...] = s[:, 1].reshape(1, 1, RBLK)


def _proj(x, W, A):
    return pl.pallas_call(
        _proj_body,
        grid=(N // RBLK,),
        in_specs=[
            pl.BlockSpec((RBLK, D), lambda i: (i, 0)),
            pl.BlockSpec((D, D), lambda i: (0, 0)),
            pl.BlockSpec((D, 2), lambda i: (0, 0)),
        ],
        out_specs=[
            pl.BlockSpec((RBLK, D), lambda i: (i, 0)),
            pl.BlockSpec((1, 1, RBLK), lambda i: (i, 0, 0)),
            pl.BlockSpec((1, 1, RBLK), lambda i: (i, 0, 0)),
        ],
        out_shape=[
            jax.ShapeDtypeStruct((N, D), jnp.float32),
            jax.ShapeDtypeStruct((N // RBLK, 1, RBLK), jnp.float32),
            jax.ShapeDtypeStruct((N // RBLK, 1, RBLK), jnp.float32),
        ],
    )(x, W, A)


def _wpass_body(s1_hbm, s2_hbm, src_hbm, dst_hbm, w_hbm, den_hbm,
                s1_v, s2_v, src_v, dst_v, w_st, den_part):
    cid = lax.axis_index("c")
    sid = lax.axis_index("s")
    wid = sid * NC + cid
    nch = jnp.where(cid == 0, NCH0, NCH1)
    st = jnp.where(cid == 0, sid * NCH0, NS * NCH0 + sid * NCH1)

    pltpu.sync_copy(s1_hbm, s1_v)
    pltpu.sync_copy(s2_hbm, s2_v)

    @pl.loop(0, N // L)
    def _(i):
        off = pl.multiple_of(i * L, L)
        den_part[pl.ds(off, L)] = jnp.zeros((L,), jnp.float32)

    @pl.loop(0, nch)
    def _(c):
        ch = st + c
        pltpu.sync_copy(src_hbm.at[ch], src_v)
        pltpu.sync_copy(dst_hbm.at[ch], dst_v)

        @pl.loop(0, RING)
        def _(b):
            for g in range(B // L):
                sv = src_v[b, pl.ds(g * L, L)]
                dv = dst_v[b, pl.ds(g * L, L)]
                e = plsc.load_gather(s1_v, [sv]) + plsc.load_gather(s2_v, [dv])
                e = jnp.where(e > 0, e, e * 0.01)
                w = jnp.exp(e)
                gid = ch * CHE + b * B + g * L + lax.iota(jnp.int32, L)
                w = jnp.where(gid < E, w, 0.0)
                w_st[b, pl.ds(g * L, L)] = w
                plsc.addupdate_scatter(den_part, [dv], w)

        pltpu.sync_copy(w_st, w_hbm.at[ch])

    pltpu.sync_copy(den_part, den_hbm.at[wid])


def _wpass(s1, s2, src_p, dst_p):
    mesh = plsc.VectorSubcoreMesh(core_axis_name="c", subcore_axis_name="s")
    kern = pl.kernel(
        _wpass_body,
        out_type=[
            jax.ShapeDtypeStruct((TOTCH, RING, B), jnp.float32),
            jax.ShapeDtypeStruct((NW, N), jnp.float32),
        ],
        mesh=mesh,
        scratch_types=[
            pltpu.VMEM((N,), jnp.float32),
            pltpu.VMEM((N,), jnp.float32),
            pltpu.VMEM((RING, B), jnp.int32),
            pltpu.VMEM((RING, B), jnp.int32),
            pltpu.VMEM((RING, B), jnp.float32),
            pltpu.VMEM((N,), jnp.float32),
        ],
        compiler_params=_sc_params(),
    )
    return kern(s1, s2, src_p, dst_p)


def _acc_body(z_hbm, w_hbm, src_hbm, dst_hbm, num_hbm,
              src_v, dst_v, w_v, rows, num_acc, gsem, ssem):
    cid = lax.axis_index("c")
    sid = lax.axis_index("s")
    nch = jnp.where(cid == 0, NCH0, NCH1)
    st = jnp.where(cid == 0, sid * NCH0, NS * NCH0 + sid * NCH1)

    @pl.loop(0, B)
    def _(r):
        for k in range(D // L):
            rows[0, r, pl.ds(k * L, L)] = jnp.zeros((L,), jnp.float32)

    for k in range(4):
        pltpu.sync_copy(rows.at[0], num_acc.at[pl.ds(sid * RPS + k * B, B)])
    pltpu.sync_copy(rows.at[0].at[pl.ds(0, RPS - 4 * B)],
                    num_acc.at[pl.ds(sid * RPS + 4 * B, RPS - 4 * B)])
    plsc.subcore_barrier()

    def _scale(b, cur):
        bvec = jnp.full((L,), b, jnp.int32)
        for r in range(B):
            wr = plsc.load_gather(w_v, [bvec, jnp.full((L,), r, jnp.int32)])
            for k in range(D // L):
                sl = pl.ds(k * L, L)
                rows[cur, r, sl] = rows[cur, r, sl] * wr

    def _scatter_wait(b, cur):
        pltpu.make_async_copy(rows.at[cur], num_acc.at[dst_v.at[b]],
                              ssem).wait()

    @pl.loop(0, nch)
    def _(c):
        ch = st + c
        pltpu.sync_copy(src_hbm.at[ch], src_v)
        pltpu.sync_copy(dst_hbm.at[ch], dst_v)
        pltpu.sync_copy(w_hbm.at[ch], w_v)

        pltpu.sync_copy(z_hbm.at[src_v.at[0]], rows.at[0])

        @pl.loop(0, RING - 1)
        def _(b):
            cur = b & 1
            nxt = (b + 1) & 1

            @pl.when(b >= 1)
            def _():
                _scatter_wait(b - 1, nxt)

            h = pltpu.async_copy(z_hbm.at[src_v.at[b + 1]], rows.at[nxt], gsem)
            _scale(b, cur)
            pltpu.async_copy(rows.at[cur], num_acc.at[dst_v.at[b]], ssem,
                             add=True)
            h.wait()

        _scale(RING - 1, (RING - 1) & 1)
        pltpu.async_copy(rows.at[(RING - 1) & 1],
                         num_acc.at[dst_v.at[RING - 1]], ssem, add=True)
        _scatter_wait(RING - 2, (RING - 2) & 1)
        _scatter_wait(RING - 1, (RING - 1) & 1)

    plsc.subcore_barrier()
    pltpu.sync_copy(num_acc.at[pl.ds(sid * RPS, RPS)],
                    num_hbm.at[cid].at[pl.ds(sid * RPS, RPS)])


def _acc(z, w, src_p, dst_p):
    mesh = plsc.VectorSubcoreMesh(core_axis_name="c", subcore_axis_name="s")
    kern = pl.kernel(
        _acc_body,
        out_type=jax.ShapeDtypeStruct((NC, NPAD, D), jnp.float32),
        mesh=mesh,
        scratch_types=[
            pltpu.VMEM((RING, B), jnp.int32),
            pltpu.VMEM((RING, B), jnp.int32),
            pltpu.VMEM((RING, B), jnp.float32),
            pltpu.VMEM((2, B, D), jnp.float32),
            pltpu.VMEM_SHARED((NPAD, D), jnp.float32),
            pltpu.SemaphoreType.DMA,
            pltpu.SemaphoreType.DMA,
        ],
        compiler_params=_sc_params(),
    )
    return kern(z, w, src_p, dst_p)


def _div_body(num_ref, den_ref, h_ref):
    nsum = num_ref[0] + num_ref[1]
    d = jnp.sum(den_ref[:, 0, 0, :], axis=0)
    dsafe = jnp.where(d > 0, d, 1.0)
    h_ref[...] = jnp.where(d[:, None] > 0, nsum / dsafe[:, None], 0.0)


def _div(num, den):
    return pl.pallas_call(
        _div_body,
        grid=(N // RBLK,),
        in_specs=[
            pl.BlockSpec((NC, RBLK, D), lambda i: (0, i, 0)),
            pl.BlockSpec((NW, 1, 1, RBLK), lambda i: (0, i, 0, 0)),
        ],
        out_specs=pl.BlockSpec((RBLK, D), lambda i: (i, 0)),
        out_shape=jax.ShapeDtypeStruct((N, D), jnp.float32),
    )(num, den)


def kernel(x, edge_index, W, attn_w):
    src = edge_index[0].astype(jnp.int32)
    dst = edge_index[1].astype(jnp.int32)
    pad = EPAD - E
    src_p = jnp.pad(src, (0, pad)).reshape(TOTCH, RING, B)
    dst_p = jnp.pad(dst, (0, pad)).reshape(TOTCH, RING, B)
    A = jnp.stack([attn_w[:D], attn_w[D:]], axis=1)
    z, s1, s2 = _proj(x, W, A)
    s1 = s1.reshape(N)
    s2 = s2.reshape(N)
    w, den = _wpass(s1, s2, src_p, dst_p)
    num = _acc(z, w, src_p, dst_p)
    den4 = den.reshape(NW, N // RBLK, 1, RBLK)
    return _div(num, den4)
